# Initial kernel scaffold; baseline (speedup 1.0000x reference)
import functools

import jax
import jax.numpy as jnp
from jax import lax
from jax.experimental import pallas as pl
from jax.experimental.pallas import tpu as pltpu
from jax.experimental.pallas import tpu_sc as plsc

N_VERT = 2048
FAN_IN = 16
BATCH = 1024
N_LAYERS = 3

NUM_CORES = 2
NUM_SUBCORES = 16
NW = NUM_CORES * NUM_SUBCORES
ROWS_PER_W = N_VERT // NW
RB = 16


def _build_a_body(src_hbm, w_hbm, a_hbm, rowbuf, src_v, w_v):
    wid = lax.axis_index("s") * NUM_CORES + lax.axis_index("c")

    def _zero(i, _):
        rowbuf[0, pl.ds(i * 16, 16)] = jnp.zeros((16,), jnp.float32)
        return ()
    lax.fori_loop(0, RB * N_VERT // 16, _zero, ())

    zeros16 = jnp.zeros((16,), jnp.float32)
    for l in range(N_LAYERS):
        for rb in range(ROWS_PER_W // RB):
            base = wid * ROWS_PER_W + rb * RB
            pltpu.sync_copy(src_hbm.at[l, pl.ds(base, RB)], src_v)
            pltpu.sync_copy(w_hbm.at[l, pl.ds(base, RB)], w_v)
            for r in range(RB):
                ridx = jnp.full((16,), r, jnp.int32)
                plsc.store_scatter(rowbuf, [ridx, src_v[r, :]], w_v[r, :])
            pltpu.sync_copy(rowbuf, a_hbm.at[l, pl.ds(base, RB)])
            for r in range(RB):
                ridx = jnp.full((16,), r, jnp.int32)
                plsc.store_scatter(rowbuf, [ridx, src_v[r, :]], zeros16)


_build_a = functools.partial(
    pl.kernel,
    out_type=jax.ShapeDtypeStruct((N_LAYERS, N_VERT, N_VERT), jnp.float32),
    mesh=plsc.VectorSubcoreMesh(core_axis_name="c", subcore_axis_name="s"),
    scratch_types=[
        pltpu.VMEM((RB, N_VERT), jnp.float32),
        pltpu.VMEM((RB, FAN_IN), jnp.int32),
        pltpu.VMEM((RB, FAN_IN), jnp.float32),
    ],
)(_build_a_body)


def _tc_body(x_ref, win_ref, bin_ref, wout_ref, bout_ref, a_hbm,
             out_ref, h_ref, g_ref, abuf, sem):
    hi = jax.lax.Precision.HIGHEST
    h0 = lax.dot_general(win_ref[...], x_ref[...], (((1,), (1,)), ((), ())),
                         precision=hi)
    h_ref[...] = jnp.maximum(h0 + bin_ref[...], 0.0)
    bufs = (h_ref, g_ref)
    for l in range(N_LAYERS):
        cp = pltpu.make_async_copy(a_hbm.at[l], abuf, sem)
        cp.start()
        cp.wait()
        src, dst = bufs[l % 2], bufs[(l + 1) % 2]
        acc = lax.dot_general(abuf[...], src[...], (((1,), (0,)), ((), ())),
                              precision=hi)
        dst[...] = jnp.maximum(acc, 0.0)
    hlast = bufs[N_LAYERS % 2]
    out = lax.dot_general(hlast[...], wout_ref[...], (((0,), (1,)), ((), ())),
                          precision=hi)
    out_ref[...] = out + bout_ref[...]


def kernel(x, W_in, b_in, w1, w2, w3, src1, src2, src3, W_out, b_out):
    src_all = jnp.stack([src1, src2, src3])
    w_all = jnp.stack([w1, w2, w3])
    a_all = _build_a(src_all, w_all)

    out_size = W_out.shape[0]
    vmem = functools.partial(pl.BlockSpec, memory_space=pltpu.MemorySpace.VMEM)
    tc = pl.pallas_call(
        _tc_body,
        out_shape=jax.ShapeDtypeStruct((BATCH, out_size), jnp.float32),
        in_specs=[
            vmem(), vmem(), vmem(), vmem(), vmem(),
            pl.BlockSpec(memory_space=pltpu.MemorySpace.ANY),
        ],
        out_specs=vmem(),
        scratch_shapes=[
            pltpu.VMEM((N_VERT, BATCH), jnp.float32),
            pltpu.VMEM((N_VERT, BATCH), jnp.float32),
            pltpu.VMEM((N_VERT, N_VERT), jnp.float32),
            pltpu.SemaphoreType.DMA,
        ],
    )
    return tc(x, W_in, b_in.reshape(N_VERT, 1), W_out,
              b_out.reshape(1, out_size), a_all)


# speedup vs baseline: 2.2588x; 2.2588x over previous
import functools

import jax
import jax.numpy as jnp
from jax import lax
from jax.experimental import pallas as pl
from jax.experimental.pallas import tpu as pltpu
from jax.experimental.pallas import tpu_sc as plsc

N_VERT = 2048
FAN_IN = 16
BATCH = 1024
N_LAYERS = 3

NUM_CORES = 2
NUM_SUBCORES = 16
NW = NUM_CORES * NUM_SUBCORES
ROWS_PER_W = N_VERT // NW
RB = 16


def _build_a_body(src_hbm, w_hbm, a_hbm, rowbuf, src_v, w_v):
    wid = lax.axis_index("s") * NUM_CORES + lax.axis_index("c")

    def _zero(i, _):
        rowbuf[pl.ds(i * 16, 16)] = jnp.zeros((16,), jnp.float32)
        return ()
    lax.fori_loop(0, RB * N_VERT // 16, _zero, ())

    zeros16 = jnp.zeros((16,), jnp.float32)
    for l in range(N_LAYERS):
        for rb in range(ROWS_PER_W // RB):
            base = l * N_VERT + wid * ROWS_PER_W + rb * RB
            pltpu.sync_copy(src_hbm.at[pl.ds(base * FAN_IN, RB * FAN_IN)],
                            src_v)
            pltpu.sync_copy(w_hbm.at[pl.ds(base * FAN_IN, RB * FAN_IN)],
                            w_v)
            for r in range(RB):
                idx = src_v[pl.ds(r * FAN_IN, FAN_IN)] + r * N_VERT
                plsc.store_scatter(rowbuf, [idx],
                                   w_v[pl.ds(r * FAN_IN, FAN_IN)])
            pltpu.sync_copy(rowbuf,
                            a_hbm.at[pl.ds(base * N_VERT, RB * N_VERT)])
            for r in range(RB):
                idx = src_v[pl.ds(r * FAN_IN, FAN_IN)] + r * N_VERT
                plsc.store_scatter(rowbuf, [idx], zeros16)


_build_a = functools.partial(
    pl.kernel,
    out_type=jax.ShapeDtypeStruct((N_LAYERS * N_VERT * N_VERT,), jnp.float32),
    mesh=plsc.VectorSubcoreMesh(core_axis_name="c", subcore_axis_name="s"),
    compiler_params=pltpu.CompilerParams(needs_layout_passes=False),
    scratch_types=[
        pltpu.VMEM((RB * N_VERT,), jnp.float32),
        pltpu.VMEM((RB * FAN_IN,), jnp.int32),
        pltpu.VMEM((RB * FAN_IN,), jnp.float32),
    ],
)(_build_a_body)


MB = 256
NB = N_VERT // MB


def _tc_body(x_ref, win_ref, bin_ref, wout_ref, bout_ref, a_hbm,
             out_ref, h_ref, g_ref, ab0, ab1, sem0, sem1):
    hi = jax.lax.Precision.HIGHEST
    for mb in range(NB):
        rows = pl.ds(mb * MB, MB)
        h0 = lax.dot_general(win_ref[rows, :], x_ref[...],
                             (((1,), (1,)), ((), ())), precision=hi)
        h_ref[rows, :] = jnp.maximum(h0 + bin_ref[rows, :], 0.0)

    bufs = (h_ref, g_ref)
    abufs = (ab0, ab1)
    sems = (sem0, sem1)
    blocks = [(l, mb) for l in range(N_LAYERS) for mb in range(NB)]

    def _copy(i):
        l, mb = blocks[i]
        return pltpu.make_async_copy(
            a_hbm.at[l, pl.ds(mb * MB, MB), :], abufs[i % 2], sems[i % 2])

    _copy(0).start()
    for i, (l, mb) in enumerate(blocks):
        if i + 1 < len(blocks):
            _copy(i + 1).start()
        _copy(i).wait()
        src, dst = bufs[l % 2], bufs[(l + 1) % 2]
        acc = lax.dot_general(abufs[i % 2][...], src[...],
                              (((1,), (0,)), ((), ())), precision=hi)
        dst[pl.ds(mb * MB, MB), :] = jnp.maximum(acc, 0.0)

    hlast = bufs[N_LAYERS % 2]
    out = lax.dot_general(hlast[...], wout_ref[...], (((0,), (1,)), ((), ())),
                          precision=hi)
    out_ref[...] = out + bout_ref[...]


def kernel(x, W_in, b_in, w1, w2, w3, src1, src2, src3, W_out, b_out):
    src_all = jnp.stack([src1, src2, src3]).reshape(-1)
    w_all = jnp.stack([w1, w2, w3]).reshape(-1)
    a_all = _build_a(src_all, w_all).reshape(N_LAYERS, N_VERT, N_VERT)

    out_size = W_out.shape[0]
    vmem = functools.partial(pl.BlockSpec, memory_space=pltpu.MemorySpace.VMEM)
    tc = pl.pallas_call(
        _tc_body,
        out_shape=jax.ShapeDtypeStruct((BATCH, out_size), jnp.float32),
        in_specs=[
            vmem(), vmem(), vmem(), vmem(), vmem(),
            pl.BlockSpec(memory_space=pltpu.MemorySpace.HBM),
        ],
        out_specs=vmem(),
        scratch_shapes=[
            pltpu.VMEM((N_VERT, BATCH), jnp.float32),
            pltpu.VMEM((N_VERT, BATCH), jnp.float32),
            pltpu.VMEM((MB, N_VERT), jnp.float32),
            pltpu.VMEM((MB, N_VERT), jnp.float32),
            pltpu.SemaphoreType.DMA,
            pltpu.SemaphoreType.DMA,
        ],
    )
    return tc(x, W_in, b_in.reshape(N_VERT, 1), W_out,
              b_out.reshape(1, out_size), a_all)


# speedup vs baseline: 4.5183x; 2.0003x over previous
import functools

import jax
import jax.numpy as jnp
from jax import lax
from jax.experimental import pallas as pl
from jax.experimental.pallas import tpu as pltpu
from jax.experimental.pallas import tpu_sc as plsc

N_VERT = 2048
FAN_IN = 16
BATCH = 1024
N_LAYERS = 3
IN_SIZE = 512

NUM_CORES = 2
NUM_SUBCORES = 16
NW = NUM_CORES * NUM_SUBCORES
ROWS_PER_W = N_VERT // NW
RB = 16


def _build_a_body(src_hbm, w_hbm, a_hbm, rowbuf, src_v, w_v):
    wid = lax.axis_index("s") * NUM_CORES + lax.axis_index("c")

    def _zero(i, _):
        rowbuf[pl.ds(i * 16, 16)] = jnp.zeros((16,), jnp.float32)
        return ()
    lax.fori_loop(0, RB * N_VERT // 16, _zero, ())

    zeros16 = jnp.zeros((16,), jnp.float32)
    for l in range(N_LAYERS):
        for rb in range(ROWS_PER_W // RB):
            base = l * N_VERT + wid * ROWS_PER_W + rb * RB
            pltpu.sync_copy(src_hbm.at[pl.ds(base * FAN_IN, RB * FAN_IN)],
                            src_v)
            pltpu.sync_copy(w_hbm.at[pl.ds(base * FAN_IN, RB * FAN_IN)],
                            w_v)
            for r in range(RB):
                idx = src_v[pl.ds(r * FAN_IN, FAN_IN)] + r * N_VERT
                plsc.store_scatter(rowbuf, [idx],
                                   w_v[pl.ds(r * FAN_IN, FAN_IN)])
            pltpu.sync_copy(rowbuf,
                            a_hbm.at[pl.ds(base * N_VERT, RB * N_VERT)])
            for r in range(RB):
                idx = src_v[pl.ds(r * FAN_IN, FAN_IN)] + r * N_VERT
                plsc.store_scatter(rowbuf, [idx], zeros16)


_build_a = functools.partial(
    pl.kernel,
    out_type=jax.ShapeDtypeStruct((N_LAYERS * N_VERT * N_VERT,), jnp.float32),
    mesh=plsc.VectorSubcoreMesh(core_axis_name="c", subcore_axis_name="s"),
    compiler_params=pltpu.CompilerParams(needs_layout_passes=False),
    scratch_types=[
        pltpu.VMEM((RB * N_VERT,), jnp.float32),
        pltpu.VMEM((RB * FAN_IN,), jnp.int32),
        pltpu.VMEM((RB * FAN_IN,), jnp.float32),
    ],
)(_build_a_body)


MB = 256
NB = N_VERT // MB


def _tc_body(x_ref, win_ref, bin_ref, wout_ref, bout_ref, a_hbm,
             out_ref, h_ref, g_ref, ab0, ab1, xb_ref, sem0, sem1):
    bf = jnp.bfloat16
    f32 = jnp.float32
    xb_ref[...] = x_ref[...].astype(bf)
    for mb in range(NB):
        rows = pl.ds(mb * MB, MB)
        h0 = lax.dot_general(win_ref[rows, :].astype(bf), xb_ref[...],
                             (((1,), (1,)), ((), ())),
                             preferred_element_type=f32)
        h_ref[rows, :] = jnp.maximum(h0 + bin_ref[rows, :], 0.0).astype(bf)

    bufs = (h_ref, g_ref)
    abufs = (ab0, ab1)
    sems = (sem0, sem1)
    blocks = [(l, mb) for l in range(N_LAYERS) for mb in range(NB)]

    def _copy(i):
        l, mb = blocks[i]
        return pltpu.make_async_copy(
            a_hbm.at[l, pl.ds(mb * MB, MB), :], abufs[i % 2], sems[i % 2])

    _copy(0).start()
    for i, (l, mb) in enumerate(blocks):
        if i + 1 < len(blocks):
            _copy(i + 1).start()
        _copy(i).wait()
        src, dst = bufs[l % 2], bufs[(l + 1) % 2]
        acc = lax.dot_general(abufs[i % 2][...].astype(bf), src[...],
                              (((1,), (0,)), ((), ())),
                              preferred_element_type=f32)
        dst[pl.ds(mb * MB, MB), :] = jnp.maximum(acc, 0.0).astype(bf)

    hlast = bufs[N_LAYERS % 2]
    out = lax.dot_general(hlast[...], wout_ref[...].astype(bf),
                          (((0,), (1,)), ((), ())),
                          preferred_element_type=f32)
    out_ref[...] = out + bout_ref[...]


def kernel(x, W_in, b_in, w1, w2, w3, src1, src2, src3, W_out, b_out):
    src_all = jnp.stack([src1, src2, src3]).reshape(-1)
    w_all = jnp.stack([w1, w2, w3]).reshape(-1)
    a_all = _build_a(src_all, w_all).reshape(N_LAYERS, N_VERT, N_VERT)

    out_size = W_out.shape[0]
    vmem = functools.partial(pl.BlockSpec, memory_space=pltpu.MemorySpace.VMEM)
    tc = pl.pallas_call(
        _tc_body,
        out_shape=jax.ShapeDtypeStruct((BATCH, out_size), jnp.float32),
        in_specs=[
            vmem(), vmem(), vmem(), vmem(), vmem(),
            pl.BlockSpec(memory_space=pltpu.MemorySpace.HBM),
        ],
        out_specs=vmem(),
        scratch_shapes=[
            pltpu.VMEM((N_VERT, BATCH), jnp.bfloat16),
            pltpu.VMEM((N_VERT, BATCH), jnp.bfloat16),
            pltpu.VMEM((MB, N_VERT), jnp.float32),
            pltpu.VMEM((MB, N_VERT), jnp.float32),
            pltpu.VMEM((BATCH, IN_SIZE), jnp.bfloat16),
            pltpu.SemaphoreType.DMA,
            pltpu.SemaphoreType.DMA,
        ],
    )
    return tc(x, W_in, b_in.reshape(N_VERT, 1), W_out,
              b_out.reshape(1, out_size), a_all)


# speedup vs baseline: 7.3092x; 1.6177x over previous
import functools

import jax
import jax.numpy as jnp
from jax import lax
from jax.experimental import pallas as pl
from jax.experimental.pallas import tpu as pltpu
from jax.experimental.pallas import tpu_sc as plsc

N_VERT = 2048
FAN_IN = 16
BATCH = 1024
N_LAYERS = 3
IN_SIZE = 512

NUM_CORES = 2
NUM_SUBCORES = 16
NW = NUM_CORES * NUM_SUBCORES
ROWS_PER_W = N_VERT // NW
RB = 16


N_BATCH = N_LAYERS * ROWS_PER_W // RB


def _build_a_body(src_hbm, w_hbm, a_hbm, rb0, rb1, src_v, w_v,
                  sem0, sem1, lsem):
    wid = lax.axis_index("s") * NUM_CORES + lax.axis_index("c")

    for l in range(N_LAYERS):
        off = (l * N_VERT + wid * ROWS_PER_W) * FAN_IN
        cnt = ROWS_PER_W * FAN_IN
        pltpu.async_copy(src_hbm.at[pl.ds(off, cnt)],
                         src_v.at[pl.ds(l * cnt, cnt)], lsem).wait()
        pltpu.async_copy(w_hbm.at[pl.ds(off, cnt)],
                         w_v.at[pl.ds(l * cnt, cnt)], lsem).wait()

    def _zero(i, _):
        rb0[pl.ds(i * 16, 16)] = jnp.zeros((16,), jnp.float32)
        rb1[pl.ds(i * 16, 16)] = jnp.zeros((16,), jnp.float32)
        return ()
    lax.fori_loop(0, RB * N_VERT // 16, _zero, ())

    bufs = (rb0, rb1)
    sems = (sem0, sem1)
    zeros16 = jnp.zeros((16,), jnp.float32)

    def _dma(i):
        l, rb = divmod(i, ROWS_PER_W // RB)
        base = l * N_VERT + wid * ROWS_PER_W + rb * RB
        return pltpu.make_async_copy(
            bufs[i % 2], a_hbm.at[pl.ds(base * N_VERT, RB * N_VERT)],
            sems[i % 2])

    def _scat(i, vals_of):
        l, rb = divmod(i, ROWS_PER_W // RB)
        buf = bufs[i % 2]
        for r in range(RB):
            o = (l * ROWS_PER_W + rb * RB + r) * FAN_IN
            idx = src_v[pl.ds(o, FAN_IN)] + r * N_VERT
            plsc.store_scatter(buf, [idx], vals_of(o))

    for i in range(N_BATCH):
        if i >= 2:
            _dma(i - 2).wait()
            _scat(i - 2, lambda o: zeros16)
        _scat(i, lambda o: w_v[pl.ds(o, FAN_IN)])
        _dma(i).start()
    _dma(N_BATCH - 2).wait()
    _dma(N_BATCH - 1).wait()


_build_a = functools.partial(
    pl.kernel,
    out_type=jax.ShapeDtypeStruct((N_LAYERS * N_VERT * N_VERT,), jnp.float32),
    mesh=plsc.VectorSubcoreMesh(core_axis_name="c", subcore_axis_name="s"),
    compiler_params=pltpu.CompilerParams(needs_layout_passes=False),
    scratch_types=[
        pltpu.VMEM((RB * N_VERT,), jnp.float32),
        pltpu.VMEM((RB * N_VERT,), jnp.float32),
        pltpu.VMEM((N_LAYERS * ROWS_PER_W * FAN_IN,), jnp.int32),
        pltpu.VMEM((N_LAYERS * ROWS_PER_W * FAN_IN,), jnp.float32),
        pltpu.SemaphoreType.DMA,
        pltpu.SemaphoreType.DMA,
        pltpu.SemaphoreType.DMA,
    ],
)(_build_a_body)


MB = 256
NB = N_VERT // MB


def _tc_body(x_ref, win_ref, bin_ref, wout_ref, bout_ref, a_hbm,
             out_ref, h_ref, g_ref, ab0, ab1, xb_ref, sem0, sem1):
    bf = jnp.bfloat16
    f32 = jnp.float32
    xb_ref[...] = x_ref[...].astype(bf)
    for mb in range(NB):
        rows = pl.ds(mb * MB, MB)
        h0 = lax.dot_general(win_ref[rows, :].astype(bf), xb_ref[...],
                             (((1,), (1,)), ((), ())),
                             preferred_element_type=f32)
        h_ref[rows, :] = jnp.maximum(h0 + bin_ref[rows, :], 0.0).astype(bf)

    bufs = (h_ref, g_ref)
    abufs = (ab0, ab1)
    sems = (sem0, sem1)
    blocks = [(l, mb) for l in range(N_LAYERS) for mb in range(NB)]

    a3 = a_hbm.reshape(N_LAYERS * NB, MB, N_VERT)

    def _copy(i):
        l, mb = blocks[i]
        return pltpu.make_async_copy(
            a3.at[l * NB + mb], abufs[i % 2], sems[i % 2])

    _copy(0).start()
    for i, (l, mb) in enumerate(blocks):
        if i + 1 < len(blocks):
            _copy(i + 1).start()
        _copy(i).wait()
        src, dst = bufs[l % 2], bufs[(l + 1) % 2]
        acc = lax.dot_general(abufs[i % 2][...].astype(bf), src[...],
                              (((1,), (0,)), ((), ())),
                              preferred_element_type=f32)
        dst[pl.ds(mb * MB, MB), :] = jnp.maximum(acc, 0.0).astype(bf)

    hlast = bufs[N_LAYERS % 2]
    out = lax.dot_general(hlast[...], wout_ref[...].astype(bf),
                          (((0,), (1,)), ((), ())),
                          preferred_element_type=f32)
    out_ref[...] = out + bout_ref[...]


def kernel(x, W_in, b_in, w1, w2, w3, src1, src2, src3, W_out, b_out):
    src_all = jnp.stack([src1, src2, src3]).reshape(-1)
    w_all = jnp.stack([w1, w2, w3]).reshape(-1)
    a_all = _build_a(src_all, w_all).reshape(N_LAYERS * N_VERT * N_VERT // 128,
                                             128)

    out_size = W_out.shape[0]
    vmem = functools.partial(pl.BlockSpec, memory_space=pltpu.MemorySpace.VMEM)
    tc = pl.pallas_call(
        _tc_body,
        out_shape=jax.ShapeDtypeStruct((BATCH, out_size), jnp.float32),
        in_specs=[
            vmem(), vmem(), vmem(), vmem(), vmem(),
            pl.BlockSpec(memory_space=pltpu.MemorySpace.HBM),
        ],
        out_specs=vmem(),
        scratch_shapes=[
            pltpu.VMEM((N_VERT, BATCH), jnp.bfloat16),
            pltpu.VMEM((N_VERT, BATCH), jnp.bfloat16),
            pltpu.VMEM((MB, N_VERT), jnp.float32),
            pltpu.VMEM((MB, N_VERT), jnp.float32),
            pltpu.VMEM((BATCH, IN_SIZE), jnp.bfloat16),
            pltpu.SemaphoreType.DMA,
            pltpu.SemaphoreType.DMA,
        ],
    )
    return tc(x, W_in, b_in.reshape(N_VERT, 1), W_out,
              b_out.reshape(1, out_size), a_all)


# speedup vs baseline: 7.5551x; 1.0336x over previous
import functools

import jax
import jax.numpy as jnp
from jax import lax
from jax.experimental import pallas as pl
from jax.experimental.pallas import tpu as pltpu
from jax.experimental.pallas import tpu_sc as plsc

N_VERT = 2048
FAN_IN = 16
BATCH = 1024
N_LAYERS = 3
IN_SIZE = 512

NUM_CORES = 2
NUM_SUBCORES = 16
NW = NUM_CORES * NUM_SUBCORES
ROWS_PER_W = N_VERT // NW
RB = 16


N_BATCH = N_LAYERS * ROWS_PER_W // RB


def _build_a_body(src_hbm, w_hbm, a_hbm, rb0, rb1, src_v, w_v,
                  sem0, sem1, lsem):
    wid = lax.axis_index("s") * NUM_CORES + lax.axis_index("c")

    for l in range(N_LAYERS):
        off = (l * N_VERT + wid * ROWS_PER_W) * FAN_IN
        cnt = ROWS_PER_W * FAN_IN
        pltpu.async_copy(src_hbm.at[pl.ds(off, cnt)],
                         src_v.at[pl.ds(l * cnt, cnt)], lsem).wait()
        pltpu.async_copy(w_hbm.at[pl.ds(off, cnt)],
                         w_v.at[pl.ds(l * cnt, cnt)], lsem).wait()

    def _zero(i, _):
        rb0[pl.ds(i * 16, 16)] = jnp.zeros((16,), jnp.float32)
        rb1[pl.ds(i * 16, 16)] = jnp.zeros((16,), jnp.float32)
        return ()
    lax.fori_loop(0, RB * N_VERT // 16, _zero, ())

    bufs = (rb0, rb1)
    sems = (sem0, sem1)
    zeros16 = jnp.zeros((16,), jnp.float32)

    def _dma(i):
        l, rb = divmod(i, ROWS_PER_W // RB)
        base = l * N_VERT + wid * ROWS_PER_W + rb * RB
        return pltpu.make_async_copy(
            bufs[i % 2], a_hbm.at[pl.ds(base * N_VERT, RB * N_VERT)],
            sems[i % 2])

    def _scat(i, vals_of):
        l, rb = divmod(i, ROWS_PER_W // RB)
        buf = bufs[i % 2]
        for r in range(RB):
            o = (l * ROWS_PER_W + rb * RB + r) * FAN_IN
            idx = src_v[pl.ds(o, FAN_IN)] + r * N_VERT
            plsc.store_scatter(buf, [idx], vals_of(o))

    for i in range(N_BATCH):
        if i >= 2:
            _dma(i - 2).wait()
            _scat(i - 2, lambda o: zeros16)
        _scat(i, lambda o: w_v[pl.ds(o, FAN_IN)])
        _dma(i).start()
    _dma(N_BATCH - 2).wait()
    _dma(N_BATCH - 1).wait()


_build_a = functools.partial(
    pl.kernel,
    out_type=jax.ShapeDtypeStruct((N_LAYERS * N_VERT * N_VERT,), jnp.float32),
    mesh=plsc.VectorSubcoreMesh(core_axis_name="c", subcore_axis_name="s"),
    compiler_params=pltpu.CompilerParams(needs_layout_passes=False),
    scratch_types=[
        pltpu.VMEM((RB * N_VERT,), jnp.float32),
        pltpu.VMEM((RB * N_VERT,), jnp.float32),
        pltpu.VMEM((N_LAYERS * ROWS_PER_W * FAN_IN,), jnp.int32),
        pltpu.VMEM((N_LAYERS * ROWS_PER_W * FAN_IN,), jnp.float32),
        pltpu.SemaphoreType.DMA,
        pltpu.SemaphoreType.DMA,
        pltpu.SemaphoreType.DMA,
    ],
)(_build_a_body)


MB = 256
NB = N_VERT // MB


def _tc_body(x_ref, win_ref, bin_ref, wout_ref, bout_ref, a_hbm,
             out_ref, h_ref, g_ref, ab0, ab1, xb_ref, sem0, sem1):
    bf = jnp.bfloat16
    f32 = jnp.float32
    xb_ref[...] = x_ref[...].astype(bf)
    for mb in range(NB):
        rows = pl.ds(mb * MB, MB)
        h0 = lax.dot_general(win_ref[rows, :].astype(bf), xb_ref[...],
                             (((1,), (1,)), ((), ())),
                             preferred_element_type=f32)
        h_ref[rows, :] = jnp.maximum(h0 + bin_ref[rows, :], 0.0).astype(bf)

    bufs = (h_ref, g_ref)
    abufs = (ab0, ab1)
    sems = (sem0, sem1)
    blocks = [(l, mb) for l in range(N_LAYERS) for mb in range(NB)]

    a3 = a_hbm.reshape(N_LAYERS * NB, MB * N_VERT // 128, 128)

    def _copy(i):
        l, mb = blocks[i]
        return pltpu.make_async_copy(
            a3.at[l * NB + mb], abufs[i % 2], sems[i % 2])

    _copy(0).start()
    for i, (l, mb) in enumerate(blocks):
        if i + 1 < len(blocks):
            _copy(i + 1).start()
        _copy(i).wait()
        src, dst = bufs[l % 2], bufs[(l + 1) % 2]
        a_blk = abufs[i % 2].reshape(MB, N_VERT)
        acc = lax.dot_general(a_blk[...].astype(bf), src[...],
                              (((1,), (0,)), ((), ())),
                              preferred_element_type=f32)
        dst[pl.ds(mb * MB, MB), :] = jnp.maximum(acc, 0.0).astype(bf)

    hlast = bufs[N_LAYERS % 2]
    out = lax.dot_general(hlast[...], wout_ref[...].astype(bf),
                          (((0,), (1,)), ((), ())),
                          preferred_element_type=f32)
    out_ref[...] = out + bout_ref[...]


def kernel(x, W_in, b_in, w1, w2, w3, src1, src2, src3, W_out, b_out):
    src_all = jnp.stack([src1, src2, src3]).reshape(-1)
    w_all = jnp.stack([w1, w2, w3]).reshape(-1)
    a_all = _build_a(src_all, w_all).reshape(N_LAYERS * N_VERT * N_VERT // 128,
                                             128)

    out_size = W_out.shape[0]
    vmem = functools.partial(pl.BlockSpec, memory_space=pltpu.MemorySpace.VMEM)
    tc = pl.pallas_call(
        _tc_body,
        out_shape=jax.ShapeDtypeStruct((BATCH, out_size), jnp.float32),
        in_specs=[
            vmem(), vmem(), vmem(), vmem(), vmem(),
            pl.BlockSpec(memory_space=pltpu.MemorySpace.HBM),
        ],
        out_specs=vmem(),
        scratch_shapes=[
            pltpu.VMEM((N_VERT, BATCH), jnp.bfloat16),
            pltpu.VMEM((N_VERT, BATCH), jnp.bfloat16),
            pltpu.VMEM((MB * N_VERT // 128, 128), jnp.float32),
            pltpu.VMEM((MB * N_VERT // 128, 128), jnp.float32),
            pltpu.VMEM((BATCH, IN_SIZE), jnp.bfloat16),
            pltpu.SemaphoreType.DMA,
            pltpu.SemaphoreType.DMA,
        ],
    )
    return tc(x, W_in, b_in.reshape(N_VERT, 1), W_out,
              b_out.reshape(1, out_size), a_all)


# speedup vs baseline: 7.7128x; 1.0209x over previous
import functools

import jax
import jax.numpy as jnp
from jax import lax
from jax.experimental import pallas as pl
from jax.experimental.pallas import tpu as pltpu
from jax.experimental.pallas import tpu_sc as plsc

N_VERT = 2048
FAN_IN = 16
BATCH = 1024
N_LAYERS = 3
IN_SIZE = 512

NUM_CORES = 2
NUM_SUBCORES = 16
NW = NUM_CORES * NUM_SUBCORES
ROWS_PER_W = N_VERT // NW
RB = 16


N_BATCH = N_LAYERS * ROWS_PER_W // RB


def _build_a_body(src_hbm, w_hbm, a_hbm, rb0, rb1, src_v, w_v,
                  sem0, sem1, lsem):
    wid = lax.axis_index("s") * NUM_CORES + lax.axis_index("c")

    for l in range(N_LAYERS):
        off = (l * N_VERT + wid * ROWS_PER_W) * FAN_IN
        cnt = ROWS_PER_W * FAN_IN
        pltpu.async_copy(src_hbm.at[pl.ds(off, cnt)],
                         src_v.at[pl.ds(l * cnt, cnt)], lsem).wait()
        pltpu.async_copy(w_hbm.at[pl.ds(off, cnt)],
                         w_v.at[pl.ds(l * cnt, cnt)], lsem).wait()

    def _zero(i, _):
        rb0[pl.ds(i * 16, 16)] = jnp.zeros((16,), jnp.float32)
        rb1[pl.ds(i * 16, 16)] = jnp.zeros((16,), jnp.float32)
        return ()
    lax.fori_loop(0, RB * N_VERT // 16, _zero, ())

    bufs = (rb0, rb1)
    sems = (sem0, sem1)
    zeros16 = jnp.zeros((16,), jnp.float32)

    def _dma(i):
        l, rb = divmod(i, ROWS_PER_W // RB)
        base = l * N_VERT + wid * ROWS_PER_W + rb * RB
        return pltpu.make_async_copy(
            bufs[i % 2], a_hbm.at[pl.ds(base * N_VERT, RB * N_VERT)],
            sems[i % 2])

    def _scat(i, vals_of):
        l, rb = divmod(i, ROWS_PER_W // RB)
        buf = bufs[i % 2]
        for r in range(RB):
            o = (l * ROWS_PER_W + rb * RB + r) * FAN_IN
            idx = src_v[pl.ds(o, FAN_IN)] + r * N_VERT
            plsc.store_scatter(buf, [idx], vals_of(o))

    for i in range(N_BATCH):
        if i >= 2:
            _dma(i - 2).wait()
            _scat(i - 2, lambda o: zeros16)
        _scat(i, lambda o: w_v[pl.ds(o, FAN_IN)])
        _dma(i).start()
    _dma(N_BATCH - 2).wait()
    _dma(N_BATCH - 1).wait()


_build_a = functools.partial(
    pl.kernel,
    out_type=jax.ShapeDtypeStruct((N_LAYERS * N_VERT * N_VERT,), jnp.float32),
    mesh=plsc.VectorSubcoreMesh(core_axis_name="c", subcore_axis_name="s"),
    compiler_params=pltpu.CompilerParams(needs_layout_passes=False),
    scratch_types=[
        pltpu.VMEM((RB * N_VERT,), jnp.float32),
        pltpu.VMEM((RB * N_VERT,), jnp.float32),
        pltpu.VMEM((N_LAYERS * ROWS_PER_W * FAN_IN,), jnp.int32),
        pltpu.VMEM((N_LAYERS * ROWS_PER_W * FAN_IN,), jnp.float32),
        pltpu.SemaphoreType.DMA,
        pltpu.SemaphoreType.DMA,
        pltpu.SemaphoreType.DMA,
    ],
)(_build_a_body)


MB = 512
NB = N_VERT // MB


def _tc_body(x_ref, win_ref, bin_ref, wout_ref, bout_ref, a_hbm,
             out_ref, h_ref, g_ref, ab0, ab1, xb_ref, sem0, sem1):
    bf = jnp.bfloat16
    f32 = jnp.float32
    xb_ref[...] = x_ref[...].astype(bf)
    for mb in range(NB):
        rows = pl.ds(mb * MB, MB)
        h0 = lax.dot_general(win_ref[rows, :].astype(bf), xb_ref[...],
                             (((1,), (1,)), ((), ())),
                             preferred_element_type=f32)
        h_ref[rows, :] = jnp.maximum(h0 + bin_ref[rows, :], 0.0).astype(bf)

    bufs = (h_ref, g_ref)
    abufs = (ab0, ab1)
    sems = (sem0, sem1)
    blocks = [(l, mb) for l in range(N_LAYERS) for mb in range(NB)]

    a3 = a_hbm.reshape(N_LAYERS * NB, MB * N_VERT // 128, 128)

    def _copy(i):
        l, mb = blocks[i]
        return pltpu.make_async_copy(
            a3.at[l * NB + mb], abufs[i % 2], sems[i % 2])

    _copy(0).start()
    for i, (l, mb) in enumerate(blocks):
        if i + 1 < len(blocks):
            _copy(i + 1).start()
        _copy(i).wait()
        src, dst = bufs[l % 2], bufs[(l + 1) % 2]
        a_blk = abufs[i % 2].reshape(MB, N_VERT)
        acc = lax.dot_general(a_blk[...].astype(bf), src[...],
                              (((1,), (0,)), ((), ())),
                              preferred_element_type=f32)
        dst[pl.ds(mb * MB, MB), :] = jnp.maximum(acc, 0.0).astype(bf)

    hlast = bufs[N_LAYERS % 2]
    out = lax.dot_general(hlast[...], wout_ref[...].astype(bf),
                          (((0,), (1,)), ((), ())),
                          preferred_element_type=f32)
    out_ref[...] = out + bout_ref[...]


def kernel(x, W_in, b_in, w1, w2, w3, src1, src2, src3, W_out, b_out):
    src_all = jnp.stack([src1, src2, src3]).reshape(-1)
    w_all = jnp.stack([w1, w2, w3]).reshape(-1)
    a_all = _build_a(src_all, w_all).reshape(N_LAYERS * N_VERT * N_VERT // 128,
                                             128)

    out_size = W_out.shape[0]
    vmem = functools.partial(pl.BlockSpec, memory_space=pltpu.MemorySpace.VMEM)
    tc = pl.pallas_call(
        _tc_body,
        out_shape=jax.ShapeDtypeStruct((BATCH, out_size), jnp.float32),
        in_specs=[
            vmem(), vmem(), vmem(), vmem(), vmem(),
            pl.BlockSpec(memory_space=pltpu.MemorySpace.HBM),
        ],
        out_specs=vmem(),
        scratch_shapes=[
            pltpu.VMEM((N_VERT, BATCH), jnp.bfloat16),
            pltpu.VMEM((N_VERT, BATCH), jnp.bfloat16),
            pltpu.VMEM((MB * N_VERT // 128, 128), jnp.float32),
            pltpu.VMEM((MB * N_VERT // 128, 128), jnp.float32),
            pltpu.VMEM((BATCH, IN_SIZE), jnp.bfloat16),
            pltpu.SemaphoreType.DMA,
            pltpu.SemaphoreType.DMA,
        ],
    )
    return tc(x, W_in, b_in.reshape(N_VERT, 1), W_out,
              b_out.reshape(1, out_size), a_all)


# speedup vs baseline: 7.8052x; 1.0120x over previous
import functools

import jax
import jax.numpy as jnp
from jax import lax
from jax.experimental import pallas as pl
from jax.experimental.pallas import tpu as pltpu
from jax.experimental.pallas import tpu_sc as plsc

N_VERT = 2048
FAN_IN = 16
BATCH = 1024
N_LAYERS = 3
IN_SIZE = 512

NUM_CORES = 2
NUM_SUBCORES = 16
NW = NUM_CORES * NUM_SUBCORES
ROWS_PER_W = N_VERT // NW
RB = 16


N_BATCH = N_LAYERS * ROWS_PER_W // RB


def _build_a_body(src_hbm, w_hbm, a_hbm, rb0, rb1, src_v, w_v,
                  sem0, sem1, lsem):
    wid = lax.axis_index("s") * NUM_CORES + lax.axis_index("c")

    for l in range(N_LAYERS):
        off = (l * N_VERT + wid * ROWS_PER_W) * FAN_IN
        cnt = ROWS_PER_W * FAN_IN
        pltpu.async_copy(src_hbm.at[pl.ds(off, cnt)],
                         src_v.at[pl.ds(l * cnt, cnt)], lsem).wait()
        pltpu.async_copy(w_hbm.at[pl.ds(off, cnt)],
                         w_v.at[pl.ds(l * cnt, cnt)], lsem).wait()

    def _zero(i, _):
        rb0[pl.ds(i * 16, 16)] = jnp.zeros((16,), jnp.float32)
        rb1[pl.ds(i * 16, 16)] = jnp.zeros((16,), jnp.float32)
        return ()
    lax.fori_loop(0, RB * N_VERT // 16, _zero, ())

    bufs = (rb0, rb1)
    sems = (sem0, sem1)
    zeros16 = jnp.zeros((16,), jnp.float32)

    def _dma(i):
        l, rb = divmod(i, ROWS_PER_W // RB)
        base = l * N_VERT + wid * ROWS_PER_W + rb * RB
        return pltpu.make_async_copy(
            bufs[i % 2], a_hbm.at[pl.ds(base * N_VERT, RB * N_VERT)],
            sems[i % 2])

    def _scat(i, vals_of):
        l, rb = divmod(i, ROWS_PER_W // RB)
        buf = bufs[i % 2]
        for r in range(RB):
            o = (l * ROWS_PER_W + rb * RB + r) * FAN_IN
            idx = src_v[pl.ds(o, FAN_IN)] + r * N_VERT
            plsc.store_scatter(buf, [idx], vals_of(o))

    for i in range(N_BATCH):
        if i >= 2:
            _dma(i - 2).wait()
            _scat(i - 2, lambda o: zeros16)
        _scat(i, lambda o: w_v[pl.ds(o, FAN_IN)])
        _dma(i).start()
    _dma(N_BATCH - 2).wait()
    _dma(N_BATCH - 1).wait()


_build_a = functools.partial(
    pl.kernel,
    out_type=jax.ShapeDtypeStruct((N_LAYERS * N_VERT * N_VERT,), jnp.float32),
    mesh=plsc.VectorSubcoreMesh(core_axis_name="c", subcore_axis_name="s"),
    compiler_params=pltpu.CompilerParams(needs_layout_passes=False),
    scratch_types=[
        pltpu.VMEM((RB * N_VERT,), jnp.float32),
        pltpu.VMEM((RB * N_VERT,), jnp.float32),
        pltpu.VMEM((N_LAYERS * ROWS_PER_W * FAN_IN,), jnp.int32),
        pltpu.VMEM((N_LAYERS * ROWS_PER_W * FAN_IN,), jnp.float32),
        pltpu.SemaphoreType.DMA,
        pltpu.SemaphoreType.DMA,
        pltpu.SemaphoreType.DMA,
    ],
)(_build_a_body)


MB = 512
NB = N_VERT // MB


def _tc_in_body(x_ref, win_ref, bin_ref, h0_ref, xb_ref):
    bf = jnp.bfloat16
    f32 = jnp.float32
    xb_ref[...] = x_ref[...].astype(bf)
    for mb in range(N_VERT // 512):
        rows = pl.ds(mb * 512, 512)
        h0 = lax.dot_general(win_ref[rows, :].astype(bf), xb_ref[...],
                             (((1,), (1,)), ((), ())),
                             preferred_element_type=f32)
        h0_ref[rows, :] = jnp.maximum(h0 + bin_ref[rows, :], 0.0).astype(bf)


def _tc_body(h0_ref, wout_ref, bout_ref, a_hbm,
             out_ref, h_ref, g_ref, ab0, ab1, sem0, sem1):
    bf = jnp.bfloat16
    f32 = jnp.float32
    bufs = (h0_ref, g_ref, h_ref)
    abufs = (ab0, ab1)
    sems = (sem0, sem1)
    blocks = [(l, mb) for l in range(N_LAYERS) for mb in range(NB)]

    a3 = a_hbm.reshape(N_LAYERS * NB, MB * N_VERT // 128, 128)

    def _copy(i):
        l, mb = blocks[i]
        return pltpu.make_async_copy(
            a3.at[l * NB + mb], abufs[i % 2], sems[i % 2])

    _copy(0).start()
    src_of = (0, 1, 2)
    dst_of = (1, 2, 1)
    for i, (l, mb) in enumerate(blocks):
        if i + 1 < len(blocks):
            _copy(i + 1).start()
        _copy(i).wait()
        src, dst = bufs[src_of[l]], bufs[dst_of[l]]
        a_blk = abufs[i % 2].reshape(MB, N_VERT)
        acc = lax.dot_general(a_blk[...].astype(bf), src[...],
                              (((1,), (0,)), ((), ())),
                              preferred_element_type=f32)
        dst[pl.ds(mb * MB, MB), :] = jnp.maximum(acc, 0.0).astype(bf)

    hlast = bufs[dst_of[N_LAYERS - 1]]
    out = lax.dot_general(hlast[...], wout_ref[...].astype(bf),
                          (((0,), (1,)), ((), ())),
                          preferred_element_type=f32)
    out_ref[...] = out + bout_ref[...]


def kernel(x, W_in, b_in, w1, w2, w3, src1, src2, src3, W_out, b_out):
    src_all = jnp.stack([src1, src2, src3]).reshape(-1)
    w_all = jnp.stack([w1, w2, w3]).reshape(-1)
    a_all = _build_a(src_all, w_all).reshape(N_LAYERS * N_VERT * N_VERT // 128,
                                             128)

    out_size = W_out.shape[0]
    vmem = functools.partial(pl.BlockSpec, memory_space=pltpu.MemorySpace.VMEM)
    tc_in = pl.pallas_call(
        _tc_in_body,
        out_shape=jax.ShapeDtypeStruct((N_VERT, BATCH), jnp.bfloat16),
        in_specs=[vmem(), vmem(), vmem()],
        out_specs=vmem(),
        scratch_shapes=[pltpu.VMEM((BATCH, IN_SIZE), jnp.bfloat16)],
    )
    h0 = tc_in(x, W_in, b_in.reshape(N_VERT, 1))
    tc = pl.pallas_call(
        _tc_body,
        out_shape=jax.ShapeDtypeStruct((BATCH, out_size), jnp.float32),
        in_specs=[
            vmem(), vmem(), vmem(),
            pl.BlockSpec(memory_space=pltpu.MemorySpace.HBM),
        ],
        out_specs=vmem(),
        scratch_shapes=[
            pltpu.VMEM((N_VERT, BATCH), jnp.bfloat16),
            pltpu.VMEM((N_VERT, BATCH), jnp.bfloat16),
            pltpu.VMEM((MB * N_VERT // 128, 128), jnp.float32),
            pltpu.VMEM((MB * N_VERT // 128, 128), jnp.float32),
            pltpu.SemaphoreType.DMA,
            pltpu.SemaphoreType.DMA,
        ],
    )
    return tc(h0, W_out, b_out.reshape(1, out_size), a_all)
